# baseline (speedup 1.0000x reference)
import functools

import jax
import jax.numpy as jnp
from jax import lax
from jax.experimental import pallas as pl
from jax.experimental.pallas import tpu as pltpu
from jax.experimental.pallas import tpu_sc as plsc

NC = 2
NS = 16
CH = 128
NBUF = 3
CHUNK_FRACS = (2, 2)


def _sc_gather(tab_flat, fl_fm, F, V, D, B):
    b_shift = B.bit_length() - 1
    assert (1 << b_shift) == B and B % CH == 0
    NW = NC * NS
    n_tot = B * F
    n_per_w = n_tot // NW
    assert n_per_w * NW == n_tot and n_per_w % CH == 0
    n_ch = n_per_w // CH
    assert n_ch % 2 == 0
    n_sl = n_ch // 2
    n_groups = (n_sl + NBUF - 1) // NBUF

    mesh = plsc.VectorSubcoreMesh(core_axis_name="c", subcore_axis_name="s")

    @functools.partial(
        pl.kernel,
        mesh=mesh,
        out_type=jax.ShapeDtypeStruct((n_tot, D), jnp.float32),
        scratch_types=(
            [pltpu.VMEM((n_per_w,), jnp.int32)]
            + [pltpu.VMEM((2 * CH, D), jnp.float32) for _ in range(NBUF)]
            + [pltpu.SemaphoreType.DMA for _ in range(2 * NBUF)]
        ),
    )
    def gather_kernel(tab_hbm, fl_hbm, out_hbm, idx_v, *bufs_and_sems):
        rows = bufs_and_sems[:NBUF]
        gsem = bufs_and_sems[NBUF:2 * NBUF]
        ssem = bufs_and_sems[2 * NBUF:]
        wid = lax.axis_index("s") * NC + lax.axis_index("c")
        base = wid * n_per_w

        pltpu.sync_copy(fl_hbm.at[pl.ds(base, n_per_w)], idx_v)

        def prep_idx(s):
            fld = lax.shift_right_logical(base + s * CH, b_shift)
            offv = jnp.full((16,), fld * V, jnp.int32)
            for k in range(CH // 16):
                sl = pl.ds(s * CH + k * 16, 16)
                idx_v[sl] = idx_v[sl] + offv

        def g_src(c):
            return tab_hbm.at[idx_v.at[pl.ds(c * CH, CH)]]

        def s_dst(s):
            return out_hbm.at[pl.ds(base + s * 2 * CH, 2 * CH)]

        def fire_gathers(s, q):
            for h in range(2):
                prep_idx(s * 2 + h)
                pltpu.async_copy(g_src(s * 2 + h),
                                 rows[q].at[pl.ds(h * CH, CH)], gsem[q])

        def wait_gathers(s, q):
            for h in range(2):
                pltpu.make_async_copy(g_src(s * 2 + h),
                                      rows[q].at[pl.ds(h * CH, CH)],
                                      gsem[q]).wait()

        for q in range(NBUF):
            fire_gathers(q, q)

        def group_body(g, carry):
            s0 = g * NBUF
            for q in range(NBUF):
                s = s0 + q

                @pl.when(s < n_sl)
                def _():
                    wait_gathers(s, q)
                    pltpu.async_copy(rows[q], s_dst(s), ssem[q])

            for q in range(NBUF):
                s = s0 + q
                sn = s + NBUF

                @pl.when(s < n_sl)
                def _():
                    pltpu.make_async_copy(rows[q], s_dst(s), ssem[q]).wait()

                @pl.when(sn < n_sl)
                def _():
                    fire_gathers(sn, q)

            return carry

        lax.fori_loop(0, n_groups, group_body, 0)

    return gather_kernel(tab_flat, fl_fm)


def _tc_linear_relu(emb3, w_bf16, bias, F, B, D, DOUT):
    BM = 256

    def mm_kernel(e_ref, w_ref, b_ref, o_ref):
        acc = jnp.zeros((BM, DOUT), jnp.float32)
        for k in range(F // 2):
            e2 = jnp.concatenate(
                [e_ref[2 * k].astype(jnp.bfloat16),
                 e_ref[2 * k + 1].astype(jnp.bfloat16)], axis=-1)
            acc = acc + jnp.dot(e2, w_ref[pl.ds(2 * k * D, 2 * D), :],
                                preferred_element_type=jnp.float32)
        if F % 2:
            acc = acc + jnp.dot(e_ref[F - 1].astype(jnp.bfloat16),
                                w_ref[pl.ds((F - 1) * D, D), :],
                                preferred_element_type=jnp.float32)
        o_ref[...] = jnp.maximum(acc + b_ref[...], 0.0)

    return pl.pallas_call(
        mm_kernel,
        grid=(B // BM,),
        in_specs=[
            pl.BlockSpec((F, BM, D), lambda i: (0, i, 0)),
            pl.BlockSpec((F * D, DOUT), lambda i: (0, 0)),
            pl.BlockSpec((1, DOUT), lambda i: (0, 0)),
        ],
        out_specs=pl.BlockSpec((BM, DOUT), lambda i: (i, 0)),
        out_shape=jax.ShapeDtypeStruct((B, DOUT), jnp.float32),
    )(emb3, w_bf16, bias)


def kernel(f_list, tables, W, b):
    F, V, D = tables.shape
    B = f_list.shape[0]
    DOUT = W.shape[1]
    tab_flat = tables.reshape(F * V, D)
    fl_t = f_list.T.astype(jnp.int32)
    w_bf16 = W.astype(jnp.bfloat16)
    bias2 = b.reshape(1, DOUT)
    outs = []
    b0 = 0
    for frac in CHUNK_FRACS:
        bc = B // frac
        fl_c = fl_t[:, b0:b0 + bc].reshape(F * bc)
        emb_c = _sc_gather(tab_flat, fl_c, F, V, D, bc)
        outs.append(_tc_linear_relu(emb_c.reshape(F, bc, D), w_bf16, bias2,
                                    F, bc, D, DOUT))
        b0 += bc
    assert b0 == B
    return jnp.concatenate(outs, axis=0)


# speedup vs baseline: 1.0274x; 1.0274x over previous
import functools

import jax
import jax.numpy as jnp
from jax import lax
from jax.experimental import pallas as pl
from jax.experimental.pallas import tpu as pltpu
from jax.experimental.pallas import tpu_sc as plsc

NC = 2
NS = 16
CH = 128
NBUF = 6
CHUNK_FRACS = (2, 2)


def _sc_gather(tab_flat, fl_fm, F, V, D, B):
    b_shift = B.bit_length() - 1
    assert (1 << b_shift) == B and B % CH == 0
    NW = NC * NS
    n_tot = B * F
    n_per_w = n_tot // NW
    assert n_per_w * NW == n_tot and n_per_w % CH == 0
    n_sl = n_per_w // CH
    n_groups = (n_sl + NBUF - 1) // NBUF

    mesh = plsc.VectorSubcoreMesh(core_axis_name="c", subcore_axis_name="s")

    @functools.partial(
        pl.kernel,
        mesh=mesh,
        out_type=jax.ShapeDtypeStruct((n_tot, D), jnp.float32),
        scratch_types=(
            [pltpu.VMEM((n_per_w,), jnp.int32)]
            + [pltpu.VMEM((CH, D), jnp.float32) for _ in range(NBUF)]
            + [pltpu.SemaphoreType.DMA for _ in range(2 * NBUF)]
        ),
    )
    def gather_kernel(tab_hbm, fl_hbm, out_hbm, idx_v, *bufs_and_sems):
        rows = bufs_and_sems[:NBUF]
        gsem = bufs_and_sems[NBUF:2 * NBUF]
        ssem = bufs_and_sems[2 * NBUF:]
        wid = lax.axis_index("s") * NC + lax.axis_index("c")
        base = wid * n_per_w

        pltpu.sync_copy(fl_hbm.at[pl.ds(base, n_per_w)], idx_v)

        def prep_idx(s):
            fld = lax.shift_right_logical(base + s * CH, b_shift)
            offv = jnp.full((16,), fld * V, jnp.int32)
            for k in range(CH // 16):
                sl = pl.ds(s * CH + k * 16, 16)
                idx_v[sl] = idx_v[sl] + offv

        def g_src(s):
            return tab_hbm.at[idx_v.at[pl.ds(s * CH, CH)]]

        def s_dst(s):
            return out_hbm.at[pl.ds(base + s * CH, CH)]

        for q in range(NBUF):
            prep_idx(q)
            pltpu.async_copy(g_src(q), rows[q], gsem[q])

        def group_body(g, carry):
            s0 = g * NBUF
            for q in range(NBUF):
                s = s0 + q

                @pl.when(s < n_sl)
                def _():
                    pltpu.make_async_copy(g_src(s), rows[q], gsem[q]).wait()
                    pltpu.async_copy(rows[q], s_dst(s), ssem[q])

            for q in range(NBUF):
                s = s0 + q
                sn = s + NBUF

                @pl.when(s < n_sl)
                def _():
                    pltpu.make_async_copy(rows[q], s_dst(s), ssem[q]).wait()

                @pl.when(sn < n_sl)
                def _():
                    prep_idx(sn)
                    pltpu.async_copy(g_src(sn), rows[q], gsem[q])

            return carry

        lax.fori_loop(0, n_groups, group_body, 0)

    return gather_kernel(tab_flat, fl_fm)


def _tc_linear_relu(emb3, w_bf16, bias, F, B, D, DOUT):
    BM = 256

    def mm_kernel(e_ref, w_ref, b_ref, o_ref):
        acc = jnp.zeros((BM, DOUT), jnp.float32)
        for k in range(F // 2):
            e2 = jnp.concatenate(
                [e_ref[2 * k].astype(jnp.bfloat16),
                 e_ref[2 * k + 1].astype(jnp.bfloat16)], axis=-1)
            acc = acc + jnp.dot(e2, w_ref[pl.ds(2 * k * D, 2 * D), :],
                                preferred_element_type=jnp.float32)
        if F % 2:
            acc = acc + jnp.dot(e_ref[F - 1].astype(jnp.bfloat16),
                                w_ref[pl.ds((F - 1) * D, D), :],
                                preferred_element_type=jnp.float32)
        o_ref[...] = jnp.maximum(acc + b_ref[...], 0.0)

    return pl.pallas_call(
        mm_kernel,
        grid=(B // BM,),
        in_specs=[
            pl.BlockSpec((F, BM, D), lambda i: (0, i, 0)),
            pl.BlockSpec((F * D, DOUT), lambda i: (0, 0)),
            pl.BlockSpec((1, DOUT), lambda i: (0, 0)),
        ],
        out_specs=pl.BlockSpec((BM, DOUT), lambda i: (i, 0)),
        out_shape=jax.ShapeDtypeStruct((B, DOUT), jnp.float32),
    )(emb3, w_bf16, bias)


def kernel(f_list, tables, W, b):
    F, V, D = tables.shape
    B = f_list.shape[0]
    DOUT = W.shape[1]
    tab_flat = tables.reshape(F * V, D)
    fl_t = f_list.T.astype(jnp.int32)
    w_bf16 = W.astype(jnp.bfloat16)
    bias2 = b.reshape(1, DOUT)
    outs = []
    b0 = 0
    for frac in CHUNK_FRACS:
        bc = B // frac
        fl_c = fl_t[:, b0:b0 + bc].reshape(F * bc)
        emb_c = _sc_gather(tab_flat, fl_c, F, V, D, bc)
        outs.append(_tc_linear_relu(emb_c.reshape(F, bc, D), w_bf16, bias2,
                                    F, bc, D, DOUT))
        b0 += bc
    assert b0 == B
    return jnp.concatenate(outs, axis=0)
